# baseline (speedup 1.0000x reference)
import functools

import jax
import jax.numpy as jnp
from jax import lax
from jax.experimental import pallas as pl
from jax.experimental.pallas import tpu as pltpu
from jax.experimental.pallas import tpu_sc as plsc

_NC, _NS, _L = 2, 16, 16
_NW = _NC * _NS
_GROUP = 16


def _sc_body(masks_hbm, gt_hbm, bounds_hbm, out_hbm,
             mbuf, gbuf, bbuf, cbuf, obuf, *, B, C, H, W):
    wid = lax.axis_index("s") * _NC + lax.axis_index("c")
    rpw = H // (_NW // B)
    b = wid // (_NW // B)
    rblk = wid % (_NW // B)

    pltpu.sync_copy(bounds_hbm.at[b], bbuf)
    xminv = bbuf[0, :]
    xmaxv = bbuf[1, :]
    yminv = bbuf[2, :]
    ymaxv = bbuf[3, :]

    one = jnp.ones((_L,), jnp.float32)
    zero = jnp.zeros((_L,), jnp.float32)
    iota = lax.broadcasted_iota(jnp.int32, (_L,), 0).astype(jnp.float32)
    for j in range(W // _L):
        colv = iota + jnp.float32(j * _L)
        inb = (colv >= xminv) & (colv < xmaxv)
        cbuf[pl.ds(j * _L, _L)] = jnp.where(inb, one, zero)

    accs = [zero] * (4 * C)

    for g in range(rpw // _GROUP):
        r0 = rblk * rpw + g * _GROUP
        pltpu.sync_copy(gt_hbm.at[b, pl.ds(r0, _GROUP), :], gbuf)
        for c in range(C):
            pltpu.sync_copy(masks_hbm.at[b, c, pl.ds(r0, _GROUP), :],
                            mbuf.at[c])

        def row_body(r, carry):
            rowf = (r0 + r).astype(jnp.float32)
            rowv = jnp.broadcast_to(rowf, (_L,))
            rin = (rowv >= yminv) & (rowv < ymaxv)
            rowinf = jnp.where(rin, one, zero)

            def vec_body(j, inner):
                b0 = j * _L
                acc = list(inner)
                colf = cbuf[pl.ds(b0, _L)]
                boxf = colf * rowinf
                nboxf = one - boxf
                gt = gbuf[r, pl.ds(b0, _L)]
                for c in range(C):
                    x = mbuf[c, r, pl.ds(b0, _L)]
                    ax = jnp.abs(x)
                    e = jnp.exp(-ax)
                    h = jnp.float32(0.04155111447344455)
                    h = h * e + jnp.float32(-0.15783837660868488)
                    h = h * e + jnp.float32(0.3065610999388668)
                    h = h * e + jnp.float32(-0.4970308426636859)
                    h = h * e + jnp.float32(0.9999449934273391)
                    sp = jnp.maximum(x, 0.0) + e * h
                    bce1 = sp - x
                    pos = gt == (c + 1)
                    posf = jnp.where(pos, one, zero)
                    negf = jnp.where(pos, nboxf, boxf)
                    acc[4 * c + 0] = acc[4 * c + 0] + bce1 * posf
                    acc[4 * c + 1] = acc[4 * c + 1] + posf
                    acc[4 * c + 2] = acc[4 * c + 2] + sp * negf
                    acc[4 * c + 3] = acc[4 * c + 3] + negf
                return tuple(acc)

            return lax.fori_loop(0, W // _L, vec_body, carry)

        accs = list(lax.fori_loop(0, _GROUP, row_body, tuple(accs)))

    for q in range(4 * C):
        obuf[q, :] = accs[q]
    pltpu.sync_copy(obuf, out_hbm.at[wid])


def sc_partials(masks, gt_masks, bounds):
    B, C, H, W = masks.shape
    mesh = plsc.VectorSubcoreMesh(core_axis_name="c", subcore_axis_name="s")
    f = functools.partial(
        pl.kernel,
        mesh=mesh,
        out_type=jax.ShapeDtypeStruct((_NW, 4 * C, _L), jnp.float32),
        scratch_types=[
            pltpu.VMEM((C, _GROUP, W), jnp.float32),
            pltpu.VMEM((_GROUP, W), jnp.int32),
            pltpu.VMEM((4, _L), jnp.float32),
            pltpu.VMEM((W,), jnp.float32),
            pltpu.VMEM((4 * C, _L), jnp.float32),
        ],
    )(functools.partial(_sc_body, B=B, C=C, H=H, W=W))
    return f(masks, gt_masks, bounds)


def kernel(skls, masks, gt_masks):
    B, C, H, W = masks.shape
    mins = jnp.min(skls, axis=1).astype(jnp.int32)
    maxs = jnp.max(skls, axis=1).astype(jnp.int32)
    x_min = jnp.clip(mins[:, 0] - 10, 0, None)
    x_max = jnp.clip(maxs[:, 0] + 10, None, W)
    y_min = jnp.clip(mins[:, 1] - 10, 0, None)
    y_max = jnp.clip(maxs[:, 1] + 10, None, H)
    bounds = jnp.stack([x_min, x_max, y_min, y_max], axis=1).astype(jnp.float32)
    bounds = jnp.broadcast_to(bounds[:, :, None], (B, 4, _L))
    parts = sc_partials(masks, gt_masks, bounds)
    sums = jnp.sum(parts, axis=(0, 2))
    loss = jnp.float32(0.0)
    for c in range(C):
        loss = loss + 0.1 * sums[4 * c + 0] / sums[4 * c + 1]
        loss = loss + 0.9 * sums[4 * c + 2] / sums[4 * c + 3]
    return loss.astype(masks.dtype)


# speedup vs baseline: 2.7972x; 2.7972x over previous
import functools

import jax
import jax.numpy as jnp
from jax import lax
from jax.experimental import pallas as pl
from jax.experimental.pallas import tpu as pltpu
from jax.experimental.pallas import tpu_sc as plsc

_NC, _NS, _L = 2, 16, 16
_NW = _NC * _NS
_B_SC = 1
_GROUP = 16


def _tc_body(skls_ref, masks_ref, gt_ref, out_ref, acc_ref, *, B, C, H, W, R):
    b = pl.program_id(0)
    nrb = H // R

    @pl.when(b == 0)
    def _init():
        acc_ref[...] = jnp.zeros_like(acc_ref)

    x_min = skls_ref[b, 0, 0]
    x_max = skls_ref[b, 0, 0]
    y_min = skls_ref[b, 0, 1]
    y_max = skls_ref[b, 0, 1]
    for j in range(1, 17):
        x_min = jnp.minimum(x_min, skls_ref[b, j, 0])
        x_max = jnp.maximum(x_max, skls_ref[b, j, 0])
        y_min = jnp.minimum(y_min, skls_ref[b, j, 1])
        y_max = jnp.maximum(y_max, skls_ref[b, j, 1])
    x_min = jnp.maximum(x_min.astype(jnp.int32) - 10, 0)
    x_max = jnp.minimum(x_max.astype(jnp.int32) + 10, W)
    y_min = jnp.maximum(y_min.astype(jnp.int32) - 10, 0)
    y_max = jnp.minimum(y_max.astype(jnp.int32) + 10, H)

    cols = jax.lax.broadcasted_iota(jnp.int32, (8, 128), 1)
    row_iota = jax.lax.broadcasted_iota(jnp.int32, (8, 128), 0)

    zeros = jnp.zeros((8, 128), jnp.float32)
    accs = [zeros] * (4 * C)
    colms = [(cols >= x_min - w * 128) & (cols < x_max - w * 128)
             for w in range(W // 128)]
    one = jnp.ones((8, 128), jnp.float32)
    for c in range(C):
        a0, a1, a2, a3 = zeros, zeros, zeros, zeros
        for s in range(R // 8):
            r0 = s * 8
            rowm = (row_iota >= y_min - r0) & (row_iota < y_max - r0)
            for w in range(W // 128):
                box = rowm & colms[w]
                boxf = jnp.where(box, 1.0, 0.0)
                nboxf = one - boxf
                gt = gt_ref[0, r0:r0 + 8, w * 128:(w + 1) * 128]
                x = masks_ref[0, c, r0:r0 + 8, w * 128:(w + 1) * 128]
                e = jnp.exp2(jnp.abs(x) * jnp.float32(-1.4426950408889634))
                sp = jnp.maximum(x, 0.0) + jnp.log(1.0 + e)
                bce1 = sp - x
                pos = gt == (c + 1)
                posf = jnp.where(pos, 1.0, 0.0)
                negf = jnp.where(pos, nboxf, boxf)
                a0 = a0 + bce1 * posf
                a1 = a1 + posf
                a2 = a2 + sp * negf
                a3 = a3 + negf
        accs[4 * c + 0] = a0
        accs[4 * c + 1] = a1
        accs[4 * c + 2] = a2
        accs[4 * c + 3] = a3

    for q in range(4 * C):
        acc_ref[q] += accs[q]

    @pl.when(b == (B - _B_SC) - 1)
    def _fin():
        for q in range(4 * C):
            out_ref[q] = jnp.sum(acc_ref[q])


def _tc_sums(skls, masks, gt_masks):
    B, C, H, W = masks.shape
    R = H
    n_tc = B - _B_SC
    return pl.pallas_call(
        functools.partial(_tc_body, B=B, C=C, H=H, W=W, R=R),
        grid=(n_tc,),
        in_specs=[
            pl.BlockSpec(memory_space=pltpu.SMEM),
            pl.BlockSpec((1, C, R, W), lambda b: (b, 0, 0, 0)),
            pl.BlockSpec((1, R, W), lambda b: (b, 0, 0)),
        ],
        out_specs=pl.BlockSpec(memory_space=pltpu.SMEM),
        out_shape=jax.ShapeDtypeStruct((4 * C,), jnp.float32),
        scratch_shapes=[pltpu.VMEM((4 * C, 8, 128), jnp.float32)],
    )(skls, masks, gt_masks)


def _sc_body(masks_hbm, gt_hbm, bounds_hbm, out_hbm,
             mbuf, gbuf, bbuf, cbuf, obuf, *, B, C, H, W):
    wid = lax.axis_index("s") * _NC + lax.axis_index("c")
    wpb = _NW // _B_SC
    rpw = H // wpb
    b = (B - _B_SC) + wid // wpb
    rblk = wid % wpb

    pltpu.sync_copy(bounds_hbm.at[b], bbuf)
    xminv = bbuf[0, :]
    xmaxv = bbuf[1, :]
    yminv = bbuf[2, :]
    ymaxv = bbuf[3, :]

    one = jnp.ones((_L,), jnp.float32)
    zero = jnp.zeros((_L,), jnp.float32)
    iota = lax.broadcasted_iota(jnp.int32, (_L,), 0).astype(jnp.float32)
    for j in range(W // _L):
        colv = iota + jnp.float32(j * _L)
        inb = (colv >= xminv) & (colv < xmaxv)
        cbuf[pl.ds(j * _L, _L)] = jnp.where(inb, one, zero)

    accs = [zero] * (4 * C)

    for g in range(rpw // _GROUP):
        r0 = rblk * rpw + g * _GROUP
        pltpu.sync_copy(gt_hbm.at[b, pl.ds(r0, _GROUP), :], gbuf)
        for c in range(C):
            pltpu.sync_copy(masks_hbm.at[b, c, pl.ds(r0, _GROUP), :],
                            mbuf.at[c])

        def row_body(r, carry):
            rowf = (r0 + r).astype(jnp.float32)
            rowv = jnp.broadcast_to(rowf, (_L,))
            rin = (rowv >= yminv) & (rowv < ymaxv)
            rowinf = jnp.where(rin, one, zero)

            def vec_body(j, inner):
                b0 = j * _L
                acc = list(inner)
                colf = cbuf[pl.ds(b0, _L)]
                boxf = colf * rowinf
                nboxf = one - boxf
                gt = gbuf[r, pl.ds(b0, _L)]
                for c in range(C):
                    x = mbuf[c, r, pl.ds(b0, _L)]
                    ax = jnp.abs(x)
                    e = jnp.exp(-ax)
                    h = jnp.float32(0.04155111447344455)
                    h = h * e + jnp.float32(-0.15783837660868488)
                    h = h * e + jnp.float32(0.3065610999388668)
                    h = h * e + jnp.float32(-0.4970308426636859)
                    h = h * e + jnp.float32(0.9999449934273391)
                    sp = jnp.maximum(x, 0.0) + e * h
                    bce1 = sp - x
                    pos = gt == (c + 1)
                    posf = jnp.where(pos, one, zero)
                    negf = jnp.where(pos, nboxf, boxf)
                    acc[4 * c + 0] = acc[4 * c + 0] + bce1 * posf
                    acc[4 * c + 1] = acc[4 * c + 1] + posf
                    acc[4 * c + 2] = acc[4 * c + 2] + sp * negf
                    acc[4 * c + 3] = acc[4 * c + 3] + negf
                return tuple(acc)

            return lax.fori_loop(0, W // _L, vec_body, carry)

        accs = list(lax.fori_loop(0, _GROUP, row_body, tuple(accs)))

    for q in range(4 * C):
        obuf[q, :] = accs[q]
    pltpu.sync_copy(obuf, out_hbm.at[wid])


def _sc_partials(masks, gt_masks, bounds):
    B, C, H, W = masks.shape
    mesh = plsc.VectorSubcoreMesh(core_axis_name="c", subcore_axis_name="s")
    f = functools.partial(
        pl.kernel,
        mesh=mesh,
        out_type=jax.ShapeDtypeStruct((_NW, 4 * C, _L), jnp.float32),
        scratch_types=[
            pltpu.VMEM((C, _GROUP, W), jnp.float32),
            pltpu.VMEM((_GROUP, W), jnp.int32),
            pltpu.VMEM((4, _L), jnp.float32),
            pltpu.VMEM((W,), jnp.float32),
            pltpu.VMEM((4 * C, _L), jnp.float32),
        ],
    )(functools.partial(_sc_body, B=B, C=C, H=H, W=W))
    return f(masks, gt_masks, bounds)


def kernel(skls, masks, gt_masks):
    B, C, H, W = masks.shape
    mins = jnp.min(skls, axis=1).astype(jnp.int32)
    maxs = jnp.max(skls, axis=1).astype(jnp.int32)
    x_min = jnp.clip(mins[:, 0] - 10, 0, None)
    x_max = jnp.clip(maxs[:, 0] + 10, None, W)
    y_min = jnp.clip(mins[:, 1] - 10, 0, None)
    y_max = jnp.clip(maxs[:, 1] + 10, None, H)
    bounds = jnp.stack([x_min, x_max, y_min, y_max], axis=1).astype(jnp.float32)
    bounds = jnp.broadcast_to(bounds[:, :, None], (B, 4, _L))
    parts = _sc_partials(masks, gt_masks, bounds)
    tc = _tc_sums(skls, masks, gt_masks)
    sums = tc + jnp.sum(parts, axis=(0, 2))
    loss = jnp.float32(0.0)
    for c in range(C):
        loss = loss + 0.1 * sums[4 * c + 0] / sums[4 * c + 1]
        loss = loss + 0.9 * sums[4 * c + 2] / sums[4 * c + 3]
    return loss.astype(masks.dtype)


# speedup vs baseline: 5.4210x; 1.9380x over previous
import functools

import jax
import jax.numpy as jnp
from jax.experimental import pallas as pl
from jax.experimental.pallas import tpu as pltpu


def _fold(q):
    return (q[:, 0:128] + q[:, 128:256]) + (q[:, 256:384] + q[:, 384:512])


def _body(skls_ref, masks_ref, gt_ref, out_ref, acc_ref, *, B, C, H, W, R):
    b = pl.program_id(0)
    rb = pl.program_id(1)
    nrb = H // R

    @pl.when((b == 0) & (rb == 0))
    def _init():
        acc_ref[...] = jnp.zeros_like(acc_ref)

    x_min = skls_ref[b, 0, 0]
    x_max = skls_ref[b, 0, 0]
    y_min = skls_ref[b, 0, 1]
    y_max = skls_ref[b, 0, 1]
    for j in range(1, 17):
        x_min = jnp.minimum(x_min, skls_ref[b, j, 0])
        x_max = jnp.maximum(x_max, skls_ref[b, j, 0])
        y_min = jnp.minimum(y_min, skls_ref[b, j, 1])
        y_max = jnp.maximum(y_max, skls_ref[b, j, 1])
    x_min = jnp.maximum(x_min.astype(jnp.int32) - 10, 0)
    x_max = jnp.minimum(x_max.astype(jnp.int32) + 10, W)
    y_min = jnp.maximum(y_min.astype(jnp.int32) - 10, 0)
    y_max = jnp.minimum(y_max.astype(jnp.int32) + 10, H)

    cols = jax.lax.broadcasted_iota(jnp.int32, (8, 128), 1)
    row_iota = jax.lax.broadcasted_iota(jnp.int32, (8, 128), 0)

    zeros = jnp.zeros((8, 128), jnp.float32)
    accs = [zeros] * (4 * C)
    base = rb * R
    colms = [(cols >= x_min - w * 128) & (cols < x_max - w * 128)
             for w in range(W // 128)]
    one = jnp.ones((8, 128), jnp.float32)
    for c in range(C):
        a0, a1, a2, a3 = zeros, zeros, zeros, zeros
        for s in range(R // 8):
            r0 = s * 8
            y_lo = y_min - (base + r0)
            y_hi = y_max - (base + r0)
            rowm = (row_iota >= y_lo) & (row_iota < y_hi)
            for w in range(W // 128):
                box = rowm & colms[w]
                boxf = jnp.where(box, 1.0, 0.0)
                gt = gt_ref[0, r0:r0 + 8, w * 128:(w + 1) * 128]
                x = masks_ref[0, c, r0:r0 + 8, w * 128:(w + 1) * 128]
                e = jnp.exp2(jnp.abs(x) * jnp.float32(-1.4426950408889634))
                sp = jnp.maximum(x, 0.0) + jnp.log(1.0 + e)
                bce1 = sp - x
                posf = jnp.where(gt == (c + 1), 1.0, 0.0)
                negf = jnp.abs(boxf - posf)
                a0 = a0 + bce1 * posf
                a1 = a1 + posf
                a2 = a2 + sp * negf
                a3 = a3 + negf
        accs[4 * c + 0] = a0
        accs[4 * c + 1] = a1
        accs[4 * c + 2] = a2
        accs[4 * c + 3] = a3

    for q in range(4 * C):
        acc_ref[q] += accs[q]

    @pl.when((b == B - 1) & (rb == nrb - 1))
    def _fin():
        loss = 0.0
        for c in range(C):
            loss += 0.1 * jnp.sum(acc_ref[4 * c + 0]) / jnp.sum(acc_ref[4 * c + 1])
            loss += 0.9 * jnp.sum(acc_ref[4 * c + 2]) / jnp.sum(acc_ref[4 * c + 3])
        out_ref[0] = loss


def kernel(skls, masks, gt_masks):
    B, C, H, W = masks.shape
    R = 512
    grid = (B, H // R) if R < H else (B, 1)
    out = pl.pallas_call(
        functools.partial(_body, B=B, C=C, H=H, W=W, R=R),
        grid=grid,
        in_specs=[
            pl.BlockSpec(memory_space=pltpu.SMEM),
            pl.BlockSpec((1, C, R, W), lambda b, r: (b, 0, r, 0)),
            pl.BlockSpec((1, R, W), lambda b, r: (b, r, 0)),
        ],
        out_specs=pl.BlockSpec(memory_space=pltpu.SMEM),
        out_shape=jax.ShapeDtypeStruct((1,), masks.dtype),
        scratch_shapes=[pltpu.VMEM((4 * C, 8, 128), jnp.float32)],
    )(skls, masks, gt_masks)
    return out[0]
